# Initial kernel scaffold; baseline (speedup 1.0000x reference)
import jax
import jax.numpy as jnp
from jax.experimental import pallas as pl


def kernel(batch, table):
    raise NotImplementedError("write your pallas kernel here")


# speedup vs baseline: 1.5609x; 1.5609x over previous
import functools

import jax
import jax.numpy as jnp
from jax import lax
from jax.experimental import pallas as pl
from jax.experimental.pallas import tpu as pltpu
from jax.experimental.pallas import tpu_sc as plsc

_B0, _B1 = 16384, 26
_D = 32
_B = _B0 * _B1
_NC, _NS = 2, 16
_NW = _NC * _NS
_BPW = _B // _NW
_CH = 1664
_NG = _BPW // _CH

_mesh = plsc.VectorSubcoreMesh(core_axis_name="c", subcore_axis_name="s")


@functools.partial(
    pl.kernel,
    mesh=_mesh,
    out_type=jax.ShapeDtypeStruct((_B, _D), jnp.float32),
    scratch_types=[
        pltpu.VMEM((_CH,), jnp.int32),
        pltpu.VMEM((_CH, _D), jnp.float32),
        pltpu.SemaphoreType.DMA,
    ],
    compiler_params=pltpu.CompilerParams(use_tc_tiling_on_sc=False),
)
def _embed(idx_hbm, table_hbm, out_hbm, idx_v, rows_v, sem):
    wid = lax.axis_index("s") * _NC + lax.axis_index("c")
    base = wid * _BPW

    def body(g, carry):
        off = base + g * _CH
        pltpu.sync_copy(idx_hbm.at[pl.ds(off, _CH)], idx_v)
        pltpu.async_copy(table_hbm.at[idx_v], rows_v, sem).wait()
        pltpu.sync_copy(rows_v, out_hbm.at[pl.ds(off, _CH)])
        return carry

    lax.fori_loop(0, _NG, body, 0)


def kernel(batch, table):
    flat = batch.reshape(_B)
    out = _embed(flat, table)
    return out.reshape(_B0, _B1, _D)


# speedup vs baseline: 1.5768x; 1.0102x over previous
import functools

import jax
import jax.numpy as jnp
from jax import lax
from jax.experimental import pallas as pl
from jax.experimental.pallas import tpu as pltpu
from jax.experimental.pallas import tpu_sc as plsc

_B0, _B1 = 16384, 26
_D = 32
_B = _B0 * _B1
_NC, _NS = 2, 16
_NW = _NC * _NS
_BPW = _B // _NW
_CH = 1664
_NG = _BPW // _CH
_NB = 2

_mesh = plsc.VectorSubcoreMesh(core_axis_name="c", subcore_axis_name="s")


@functools.partial(
    pl.kernel,
    mesh=_mesh,
    out_type=jax.ShapeDtypeStruct((_B, _D), jnp.float32),
    scratch_types=[
        pltpu.VMEM((_NB, _CH), jnp.int32),
        pltpu.VMEM((_NB, _CH, _D), jnp.float32),
        pltpu.SemaphoreType.DMA,
        pltpu.SemaphoreType.DMA,
        pltpu.SemaphoreType.DMA,
        pltpu.SemaphoreType.DMA,
        pltpu.SemaphoreType.DMA,
        pltpu.SemaphoreType.DMA,
    ],
    compiler_params=pltpu.CompilerParams(use_tc_tiling_on_sc=False),
)
def _embed(idx_hbm, table_hbm, out_hbm, idx_v, rows_v,
           sem_i0, sem_i1, sem_g0, sem_g1, sem_o0, sem_o1):
    sem_i = (sem_i0, sem_i1)
    sem_g = (sem_g0, sem_g1)
    sem_o = (sem_o0, sem_o1)
    wid = lax.axis_index("s") * _NC + lax.axis_index("c")
    base = wid * _BPW

    idx_h = [None] * _NG
    gat_h = [None] * _NG
    out_h = [None] * _NG

    def load_idx(g):
        b = g % _NB
        return pltpu.async_copy(
            idx_hbm.at[pl.ds(base + g * _CH, _CH)], idx_v.at[b], sem_i[b])

    for g in range(_NB):
        idx_h[g] = load_idx(g)

    for g in range(_NG):
        b = g % _NB
        idx_h[g].wait()
        if g >= _NB:
            out_h[g - _NB].wait()
        gat_h[g] = pltpu.async_copy(
            table_hbm.at[idx_v.at[b]], rows_v.at[b], sem_g[b])
        if g >= 1:
            p = g - 1
            bp = p % _NB
            gat_h[p].wait()
            out_h[p] = pltpu.async_copy(
                rows_v.at[bp], out_hbm.at[pl.ds(base + p * _CH, _CH)],
                sem_o[bp])
            if p + _NB < _NG:
                idx_h[p + _NB] = load_idx(p + _NB)

    g = _NG - 1
    b = g % _NB
    gat_h[g].wait()
    out_h[g] = pltpu.async_copy(
        rows_v.at[b], out_hbm.at[pl.ds(base + g * _CH, _CH)], sem_o[b])
    out_h[g - 1].wait()
    out_h[g].wait()


def kernel(batch, table):
    flat = batch.reshape(_B)
    out = _embed(flat, table)
    return out.reshape(_B0, _B1, _D)
